# baseline (speedup 1.0000x reference)
import jax
import jax.numpy as jnp
from jax import lax
from jax.experimental import pallas as pl
from jax.experimental.pallas import tpu as pltpu

B, T, R = 16, 100, 200
D, H, DH, FF = 128, 4, 32, 2048
HID, V, S, NQ, NL = 4096, 32000, 512, 8, 2
EPS = 1e-5


def _shift_prev(x):
    return jnp.concatenate([jnp.zeros_like(x[:, :1, :]), x[:, :-1, :]], axis=1)


def _shift_next(x):
    return jnp.concatenate([x[:, 1:, :], jnp.zeros_like(x[:, :1, :])], axis=1)


def _conv_bn_kernel(bold_ref, w1a_ref, w1b_ref, b1a_ref, b1b_ref,
                    g1a_ref, g1b_ref, h1a_ref, h1b_ref,
                    w2a_ref, w2b_ref, c2b_ref, g2_ref, h2_ref, out_ref):
    x = jnp.nan_to_num(bold_ref[...])
    xp, xn = _shift_prev(x), _shift_next(x)
    a = w1a_ref[0] * xp + w1a_ref[1] * x + w1a_ref[2] * xn + b1a_ref[...]
    b = w1b_ref[0] * xp + w1b_ref[1] * x + w1b_ref[2] * xn + b1b_ref[...]

    def bn(y, g, h):
        m = jnp.mean(y, axis=(0, 1), keepdims=True)
        v = jnp.mean((y - m) ** 2, axis=(0, 1), keepdims=True)
        return (y - m) * lax.rsqrt(v + EPS) * g + h

    a = jnp.maximum(bn(a, g1a_ref[...], h1a_ref[...]), 0.0)
    b = jnp.maximum(bn(b, g1b_ref[...], h1b_ref[...]), 0.0)
    ap, an = _shift_prev(a), _shift_next(a)
    bp, bnx = _shift_prev(b), _shift_next(b)
    y = (w2a_ref[0] * ap + w2a_ref[1] * a + w2a_ref[2] * an
         + w2b_ref[0] * bp + w2b_ref[1] * b + w2b_ref[2] * bnx + c2b_ref[...])
    out_ref[...] = jnp.maximum(bn(y, g2_ref[...], h2_ref[...]), 0.0)


def _ln(x, g, h):
    m = jnp.mean(x, axis=-1, keepdims=True)
    v = jnp.mean((x - m) ** 2, axis=-1, keepdims=True)
    return (x - m) * lax.rsqrt(v + EPS) * g + h


def _softmax(x):
    m = jnp.max(x, axis=-1, keepdims=True)
    e = jnp.exp(x - m)
    return e / jnp.sum(e, axis=-1, keepdims=True)


def _dot_t(x, w):
    return lax.dot_general(x.astype(jnp.bfloat16), w.astype(jnp.bfloat16),
                           (((1,), (1,)), ((), ())),
                           preferred_element_type=jnp.float32)


def _dot(x, w):
    return lax.dot_general(x.astype(jnp.bfloat16), w.astype(jnp.bfloat16),
                           (((1,), (0,)), ((), ())),
                           preferred_element_type=jnp.float32)


_EG = 8


def _dot_tb(x, w):
    return lax.dot_general(x.astype(jnp.bfloat16), w.astype(jnp.bfloat16),
                           (((1,), (1,)), ((), ())),
                           preferred_element_type=jnp.float32)


def _encoder_kernel(x2_ref, tpw_ref, tpb_ref, wq_ref, wqb_ref, wk_ref, wkb_ref,
                    ln1g_ref, ln1b_ref, qkvw_ref, qkvb_ref, outw_ref, outb_ref,
                    ln2g_ref, ln2b_ref, ff1w_ref, ff1b_ref, ff2w_ref, ff2b_ref,
                    pw_ref, pb_ref, lg_ref, lb_ref, qt_ref, o_ref):
    G = _EG
    tpw = tpw_ref[...].astype(jnp.bfloat16)
    hs = [lax.dot_general(x2_ref[g].astype(jnp.bfloat16), tpw,
                          (((0,), (1,)), ((), ())),
                          preferred_element_type=jnp.float32)
          for g in range(G)]
    h = jnp.concatenate(hs, axis=0) + tpb_ref[...]
    q = _dot_tb(h, wq_ref[...]) + wqb_ref[...]
    k = _dot_tb(h, wk_ref[...]) + wkb_ref[...]
    zs = []
    for g in range(G):
        sg = slice(g * R, (g + 1) * R)
        adj = _softmax(_dot_tb(q[sg], k[sg]) * (D ** -0.5))
        zs.append(_dot(adj, h[sg]))
    z = jnp.concatenate(zs, axis=0)
    for l in range(NL):
        y = _ln(z, ln1g_ref[l], ln1b_ref[l])
        qkv = _dot_tb(y, qkvw_ref[l]) + qkvb_ref[l]
        os_ = []
        for g in range(G):
            sg = slice(g * R, (g + 1) * R)
            qkv_g = qkv[sg]
            for hh in range(H):
                qs = slice(hh * DH, (hh + 1) * DH)
                ks = slice(D + hh * DH, D + (hh + 1) * DH)
                vs = slice(2 * D + hh * DH, 2 * D + (hh + 1) * DH)
                s = _dot_t(qkv_g[:, qs], qkv_g[:, ks]) * (DH ** -0.5)
                os_.append(_dot(_softmax(s), qkv_g[:, vs]))
        o = jnp.concatenate(
            [jnp.concatenate(os_[g * H:(g + 1) * H], axis=1)
             for g in range(G)], axis=0)
        z = z + _dot_tb(o, outw_ref[l]) + outb_ref[l]
        y2 = _ln(z, ln2g_ref[l], ln2b_ref[l])
        f = jnp.maximum(_dot_tb(y2, ff1w_ref[l]) + ff1b_ref[l], 0.0)
        z = z + _dot_tb(f, ff2w_ref[l]) + ff2b_ref[l]
    qt = qt_ref[0]
    ctxs = []
    for g in range(G):
        sg = slice(g * R, (g + 1) * R)
        att = _softmax(_dot_tb(qt, z[sg]) * (D ** -0.5))
        ctxs.append(_dot(att, z[sg]))
    ctx = jnp.concatenate(ctxs, axis=0)
    br = _ln(_dot_tb(ctx, pw_ref[...]) + pb_ref[...],
             lg_ref[...], lb_ref[...])
    for g in range(G):
        o_ref[g] = br[g * NQ:(g + 1) * NQ]


def _gather_kernel(ids_ref, brain_ref, table_ref, out_ref, sem_r):
    b = pl.program_id(0)
    out_ref[0, :NQ, :] = brain_ref[0]
    for s in range(S):
        tok = ids_ref[b, s]
        pltpu.make_async_copy(
            table_ref.at[tok], out_ref.at[0, NQ + s], sem_r).start()
    wait_cp = pltpu.make_async_copy(
        table_ref.at[0], out_ref.at[0, NQ], sem_r)
    for s in range(S):
        wait_cp.wait()


def kernel(bold, input_ids, attention_mask, labels,
           conv1_w, conv1_b, bn1_g, bn1_b, conv2_w, conv2_b, bn2_g, bn2_b,
           tproj_w, tproj_b, wq_w, wq_b, wk_w, wk_b,
           enc_ln1_g, enc_ln1_b, enc_qkv_w, enc_qkv_b, enc_out_w, enc_out_b,
           enc_ln2_g, enc_ln2_b, enc_ff1_w, enc_ff1_b, enc_ff2_w, enc_ff2_b,
           proj_w, proj_b, lnf_g, lnf_b, query_tokens, embed_table):
    f32 = jnp.float32
    w1 = conv1_w.reshape(R, 2, 3)
    w1a = w1[:, 0, :].transpose(1, 0).reshape(3, 1, R)
    w1b = w1[:, 1, :].transpose(1, 0).reshape(3, 1, R)
    b1 = conv1_b.reshape(R, 2)
    b1a, b1b = b1[:, 0].reshape(1, 1, R), b1[:, 1].reshape(1, 1, R)
    g1 = bn1_g.reshape(R, 2)
    g1a, g1b = g1[:, 0].reshape(1, 1, R), g1[:, 1].reshape(1, 1, R)
    h1 = bn1_b.reshape(R, 2)
    h1a, h1b = h1[:, 0].reshape(1, 1, R), h1[:, 1].reshape(1, 1, R)
    w2a = conv2_w[:, 0, :].transpose(1, 0).reshape(3, 1, R)
    w2b = conv2_w[:, 1, :].transpose(1, 0).reshape(3, 1, R)
    c2b = conv2_b.reshape(1, 1, R)
    g2, h2 = bn2_g.reshape(1, 1, R), bn2_b.reshape(1, 1, R)

    x2 = pl.pallas_call(
        _conv_bn_kernel,
        out_shape=jax.ShapeDtypeStruct((B, T, R), f32),
        name="conv_bn",
    )(bold, w1a, w1b, b1a, b1b, g1a, g1b, h1a, h1b, w2a, w2b, c2b, g2, h2)

    full = lambda shape: pl.BlockSpec(shape, lambda b: (0,) * len(shape))
    enc_in_specs = [
        pl.BlockSpec((_EG, T, R), lambda b: (b, 0, 0)),
        full((D, T)), full((1, D)),
        full((D, D)), full((1, D)), full((D, D)), full((1, D)),
        full((NL, D)), full((NL, D)),
        full((NL, 3 * D, D)), full((NL, 3 * D)),
        full((NL, D, D)), full((NL, D)),
        full((NL, D)), full((NL, D)),
        full((NL, FF, D)), full((NL, FF)),
        full((NL, D, FF)), full((NL, D)),
        full((HID, D)), full((1, HID)), full((1, HID)), full((1, HID)),
        full((1, NQ, D)),
    ]
    brain = pl.pallas_call(
        _encoder_kernel,
        grid=(B // _EG,),
        in_specs=enc_in_specs,
        out_specs=pl.BlockSpec((_EG, NQ, HID), lambda b: (b, 0, 0)),
        out_shape=jax.ShapeDtypeStruct((B, NQ, HID), f32),
        compiler_params=pltpu.CompilerParams(
            dimension_semantics=("parallel",),
        ),
        name="encoder",
    )(x2, tproj_w, tproj_b.reshape(1, D),
      wq_w, wq_b.reshape(1, D), wk_w, wk_b.reshape(1, D),
      enc_ln1_g, enc_ln1_b, enc_qkv_w, enc_qkv_b, enc_out_w, enc_out_b,
      enc_ln2_g, enc_ln2_b, enc_ff1_w, enc_ff1_b, enc_ff2_w, enc_ff2_b,
      proj_w, proj_b.reshape(1, HID), lnf_g.reshape(1, HID),
      lnf_b.reshape(1, HID), query_tokens)

    inputs_embeds = pl.pallas_call(
        _gather_kernel,
        grid_spec=pltpu.PrefetchScalarGridSpec(
            num_scalar_prefetch=1,
            grid=(B,),
            in_specs=[pl.BlockSpec((1, NQ, HID), lambda b, ids: (b, 0, 0)),
                      pl.BlockSpec(memory_space=pl.ANY)],
            out_specs=pl.BlockSpec((1, NQ + S, HID), lambda b, ids: (b, 0, 0)),
            scratch_shapes=[pltpu.SemaphoreType.DMA],
        ),
        out_shape=jax.ShapeDtypeStruct((B, NQ + S, HID), f32),
        compiler_params=pltpu.CompilerParams(
            dimension_semantics=("arbitrary",),
            vmem_limit_bytes=50 * 1024 * 1024,
        ),
        name="gather_concat",
    )(input_ids.astype(jnp.int32), brain, embed_table)

    full_mask = jnp.concatenate(
        [jnp.ones((B, NQ), attention_mask.dtype), attention_mask], axis=1)
    full_labels = jnp.concatenate(
        [jnp.full((B, NQ), -100, labels.dtype), labels], axis=1)
    return inputs_embeds, full_mask, full_labels


# speedup vs baseline: 1.2231x; 1.2231x over previous
import jax
import jax.numpy as jnp
from jax import lax
from jax.experimental import pallas as pl
from jax.experimental.pallas import tpu as pltpu

B, T, R = 16, 100, 200
D, H, DH, FF = 128, 4, 32, 2048
HID, V, S, NQ, NL = 4096, 32000, 512, 8, 2
EPS = 1e-5


def _shift_prev(x):
    return jnp.concatenate([jnp.zeros_like(x[:, :1, :]), x[:, :-1, :]], axis=1)


def _shift_next(x):
    return jnp.concatenate([x[:, 1:, :], jnp.zeros_like(x[:, :1, :])], axis=1)


def _conv_bn_kernel(bold_ref, w1a_ref, w1b_ref, b1a_ref, b1b_ref,
                    g1a_ref, g1b_ref, h1a_ref, h1b_ref,
                    w2a_ref, w2b_ref, c2b_ref, g2_ref, h2_ref, out_ref):
    x = jnp.nan_to_num(bold_ref[...])
    xp, xn = _shift_prev(x), _shift_next(x)
    a = w1a_ref[0] * xp + w1a_ref[1] * x + w1a_ref[2] * xn + b1a_ref[...]
    b = w1b_ref[0] * xp + w1b_ref[1] * x + w1b_ref[2] * xn + b1b_ref[...]

    def bn(y, g, h):
        m = jnp.mean(y, axis=(0, 1), keepdims=True)
        v = jnp.mean((y - m) ** 2, axis=(0, 1), keepdims=True)
        return (y - m) * lax.rsqrt(v + EPS) * g + h

    a = jnp.maximum(bn(a, g1a_ref[...], h1a_ref[...]), 0.0)
    b = jnp.maximum(bn(b, g1b_ref[...], h1b_ref[...]), 0.0)
    ap, an = _shift_prev(a), _shift_next(a)
    bp, bnx = _shift_prev(b), _shift_next(b)
    y = (w2a_ref[0] * ap + w2a_ref[1] * a + w2a_ref[2] * an
         + w2b_ref[0] * bp + w2b_ref[1] * b + w2b_ref[2] * bnx + c2b_ref[...])
    out_ref[...] = jnp.maximum(bn(y, g2_ref[...], h2_ref[...]), 0.0)


def _ln(x, g, h):
    m = jnp.mean(x, axis=-1, keepdims=True)
    v = jnp.mean((x - m) ** 2, axis=-1, keepdims=True)
    return (x - m) * lax.rsqrt(v + EPS) * g + h


def _softmax(x):
    m = jnp.max(x, axis=-1, keepdims=True)
    e = jnp.exp(x - m)
    return e / jnp.sum(e, axis=-1, keepdims=True)


def _dot_t(x, w):
    return lax.dot_general(x.astype(jnp.bfloat16), w.astype(jnp.bfloat16),
                           (((1,), (1,)), ((), ())),
                           preferred_element_type=jnp.float32)


def _dot(x, w):
    return lax.dot_general(x.astype(jnp.bfloat16), w.astype(jnp.bfloat16),
                           (((1,), (0,)), ((), ())),
                           preferred_element_type=jnp.float32)


_EG = 8
DHP = 128
BF = jnp.bfloat16


def _mm_t(a, b):
    return lax.dot_general(a, b, (((1,), (1,)), ((), ())),
                           preferred_element_type=jnp.float32)


def _mm(a, b):
    return lax.dot_general(a, b, (((1,), (0,)), ((), ())),
                           preferred_element_type=jnp.float32)


def _encoder_kernel(x2_ref, tpw_ref, tpb_ref, wq_ref, wqb_ref, wk_ref, wkb_ref,
                    ln1g_ref, ln1b_ref, qw_ref, qb_ref, kw_ref, kb_ref,
                    vw_ref, vb_ref, outw_ref, outb_ref,
                    ln2g_ref, ln2b_ref, ff1w_ref, ff1b_ref, ff2w_ref, ff2b_ref,
                    pw_ref, pb_ref, lg_ref, lb_ref, qt_ref, o_ref):
    G = _EG
    scale_a = DH ** -0.5
    tpw = tpw_ref[...].astype(BF)
    hs = [lax.dot_general(x2_ref[g].astype(BF), tpw,
                          (((0,), (1,)), ((), ())),
                          preferred_element_type=jnp.float32)
          for g in range(G)]
    h = jnp.concatenate(hs, axis=0) + tpb_ref[...]
    hb = h.astype(BF)
    q = (_mm_t(hb, wq_ref[...].astype(BF)) + wqb_ref[...]).astype(BF)
    k = (_mm_t(hb, wk_ref[...].astype(BF)) + wkb_ref[...]).astype(BF)
    zs = []
    for g in range(G):
        sg = slice(g * R, (g + 1) * R)
        adj = _softmax(_mm_t(q[sg], k[sg]) * (D ** -0.5)).astype(BF)
        zs.append(_mm(adj, hb[sg]))
    z = jnp.concatenate(zs, axis=0)
    for l in range(NL):
        yb = _ln(z, ln1g_ref[l], ln1b_ref[l]).astype(BF)
        qp = (_mm_t(yb, qw_ref[l].astype(BF)) + qb_ref[l]).astype(BF)
        kp = (_mm_t(yb, kw_ref[l].astype(BF)) + kb_ref[l]).astype(BF)
        vp = (_mm_t(yb, vw_ref[l].astype(BF)) + vb_ref[l]).astype(BF)
        os_ = []
        for g in range(G):
            sg = slice(g * R, (g + 1) * R)
            for hh in range(H):
                hsl = slice(hh * DHP, (hh + 1) * DHP)
                s = _mm_t(qp[sg, hsl], kp[sg, hsl]) * scale_a
                p = _softmax(s).astype(BF)
                os_.append(_mm(p, vp[sg, hsl]).astype(BF))
        o = jnp.concatenate(
            [jnp.concatenate(os_[g * H:(g + 1) * H], axis=1)
             for g in range(G)], axis=0)
        z = z + _mm_t(o, outw_ref[l].astype(BF)) + outb_ref[l]
        y2b = _ln(z, ln2g_ref[l], ln2b_ref[l]).astype(BF)
        f = jnp.maximum(_mm_t(y2b, ff1w_ref[l].astype(BF)) + ff1b_ref[l], 0.0)
        z = z + _mm_t(f.astype(BF), ff2w_ref[l].astype(BF)) + ff2b_ref[l]
    qtb = qt_ref[0].astype(BF)
    zb = z.astype(BF)
    ctxs = []
    for g in range(G):
        sg = slice(g * R, (g + 1) * R)
        att = _softmax(_mm_t(qtb, zb[sg]) * (D ** -0.5)).astype(BF)
        ctxs.append(_mm(att, zb[sg]))
    ctx = jnp.concatenate(ctxs, axis=0).astype(BF)
    br = _ln(_mm_t(ctx, pw_ref[...].astype(BF)) + pb_ref[...],
             lg_ref[...], lb_ref[...])
    for g in range(G):
        o_ref[g] = br[g * NQ:(g + 1) * NQ]


def _gather_kernel(ids_ref, brain_ref, table_ref, out_ref, sem_r):
    b = pl.program_id(0)
    out_ref[0, :NQ, :] = brain_ref[0]
    for s in range(S):
        tok = ids_ref[b, s]
        pltpu.make_async_copy(
            table_ref.at[tok], out_ref.at[0, NQ + s], sem_r).start()
    wait_cp = pltpu.make_async_copy(
        table_ref.at[0], out_ref.at[0, NQ], sem_r)
    for s in range(S):
        wait_cp.wait()


def kernel(bold, input_ids, attention_mask, labels,
           conv1_w, conv1_b, bn1_g, bn1_b, conv2_w, conv2_b, bn2_g, bn2_b,
           tproj_w, tproj_b, wq_w, wq_b, wk_w, wk_b,
           enc_ln1_g, enc_ln1_b, enc_qkv_w, enc_qkv_b, enc_out_w, enc_out_b,
           enc_ln2_g, enc_ln2_b, enc_ff1_w, enc_ff1_b, enc_ff2_w, enc_ff2_b,
           proj_w, proj_b, lnf_g, lnf_b, query_tokens, embed_table):
    f32 = jnp.float32
    w1 = conv1_w.reshape(R, 2, 3)
    w1a = w1[:, 0, :].transpose(1, 0).reshape(3, 1, R)
    w1b = w1[:, 1, :].transpose(1, 0).reshape(3, 1, R)
    b1 = conv1_b.reshape(R, 2)
    b1a, b1b = b1[:, 0].reshape(1, 1, R), b1[:, 1].reshape(1, 1, R)
    g1 = bn1_g.reshape(R, 2)
    g1a, g1b = g1[:, 0].reshape(1, 1, R), g1[:, 1].reshape(1, 1, R)
    h1 = bn1_b.reshape(R, 2)
    h1a, h1b = h1[:, 0].reshape(1, 1, R), h1[:, 1].reshape(1, 1, R)
    w2a = conv2_w[:, 0, :].transpose(1, 0).reshape(3, 1, R)
    w2b = conv2_w[:, 1, :].transpose(1, 0).reshape(3, 1, R)
    c2b = conv2_b.reshape(1, 1, R)
    g2, h2 = bn2_g.reshape(1, 1, R), bn2_b.reshape(1, 1, R)

    x2 = pl.pallas_call(
        _conv_bn_kernel,
        out_shape=jax.ShapeDtypeStruct((B, T, R), f32),
        name="conv_bn",
    )(bold, w1a, w1b, b1a, b1b, g1a, g1b, h1a, h1b, w2a, w2b, c2b, g2, h2)

    full = lambda shape: pl.BlockSpec(shape, lambda b: (0,) * len(shape))
    enc_in_specs = [
        pl.BlockSpec((_EG, T, R), lambda b: (b, 0, 0)),
        full((D, T)), full((1, D)),
        full((D, D)), full((1, D)), full((D, D)), full((1, D)),
        full((NL, D)), full((NL, D)),
        full((NL, H * DHP, D)), full((NL, H * DHP)),
        full((NL, H * DHP, D)), full((NL, H * DHP)),
        full((NL, H * DHP, D)), full((NL, H * DHP)),
        full((NL, D, H * DHP)), full((NL, D)),
        full((NL, D)), full((NL, D)),
        full((NL, FF, D)), full((NL, FF)),
        full((NL, D, FF)), full((NL, D)),
        full((HID, D)), full((1, HID)), full((1, HID)), full((1, HID)),
        full((1, NQ, D)),
    ]
    def _pad_heads_rows(w):
        return jnp.pad(w.reshape(NL, H, DH, D),
                       ((0, 0), (0, 0), (0, DHP - DH), (0, 0))
                       ).reshape(NL, H * DHP, D)

    def _pad_heads_bias(bv):
        return jnp.pad(bv.reshape(NL, H, DH),
                       ((0, 0), (0, 0), (0, DHP - DH))).reshape(NL, H * DHP)

    qw_p = _pad_heads_rows(enc_qkv_w[:, :D, :])
    kw_p = _pad_heads_rows(enc_qkv_w[:, D:2 * D, :])
    vw_p = _pad_heads_rows(enc_qkv_w[:, 2 * D:, :])
    qb_p = _pad_heads_bias(enc_qkv_b[:, :D])
    kb_p = _pad_heads_bias(enc_qkv_b[:, D:2 * D])
    vb_p = _pad_heads_bias(enc_qkv_b[:, 2 * D:])
    outw_p = jnp.pad(enc_out_w.reshape(NL, D, H, DH),
                     ((0, 0), (0, 0), (0, 0), (0, DHP - DH))
                     ).reshape(NL, D, H * DHP)

    brain = pl.pallas_call(
        _encoder_kernel,
        grid=(B // _EG,),
        in_specs=enc_in_specs,
        out_specs=pl.BlockSpec((_EG, NQ, HID), lambda b: (b, 0, 0)),
        out_shape=jax.ShapeDtypeStruct((B, NQ, HID), f32),
        compiler_params=pltpu.CompilerParams(
            dimension_semantics=("parallel",),
        ),
        name="encoder",
    )(x2, tproj_w, tproj_b.reshape(1, D),
      wq_w, wq_b.reshape(1, D), wk_w, wk_b.reshape(1, D),
      enc_ln1_g, enc_ln1_b, qw_p, qb_p, kw_p, kb_p, vw_p, vb_p,
      outw_p, enc_out_b,
      enc_ln2_g, enc_ln2_b, enc_ff1_w, enc_ff1_b, enc_ff2_w, enc_ff2_b,
      proj_w, proj_b.reshape(1, HID), lnf_g.reshape(1, HID),
      lnf_b.reshape(1, HID), query_tokens)

    inputs_embeds = pl.pallas_call(
        _gather_kernel,
        grid_spec=pltpu.PrefetchScalarGridSpec(
            num_scalar_prefetch=1,
            grid=(B,),
            in_specs=[pl.BlockSpec((1, NQ, HID), lambda b, ids: (b, 0, 0)),
                      pl.BlockSpec(memory_space=pl.ANY)],
            out_specs=pl.BlockSpec((1, NQ + S, HID), lambda b, ids: (b, 0, 0)),
            scratch_shapes=[pltpu.SemaphoreType.DMA],
        ),
        out_shape=jax.ShapeDtypeStruct((B, NQ + S, HID), f32),
        compiler_params=pltpu.CompilerParams(
            dimension_semantics=("arbitrary",),
            vmem_limit_bytes=50 * 1024 * 1024,
        ),
        name="gather_concat",
    )(input_ids.astype(jnp.int32), brain, embed_table)

    full_mask = jnp.concatenate(
        [jnp.ones((B, NQ), attention_mask.dtype), attention_mask], axis=1)
    full_labels = jnp.concatenate(
        [jnp.full((B, NQ), -100, labels.dtype), labels], axis=1)
    return inputs_embeds, full_mask, full_labels


# speedup vs baseline: 1.3032x; 1.0654x over previous
import jax
import jax.numpy as jnp
from jax import lax
from jax.experimental import pallas as pl
from jax.experimental.pallas import tpu as pltpu

B, T, R = 16, 100, 200
D, H, DH, FF = 128, 4, 32, 2048
HID, V, S, NQ, NL = 4096, 32000, 512, 8, 2
EPS = 1e-5


def _shift_prev(x):
    return jnp.concatenate([jnp.zeros_like(x[:, :1, :]), x[:, :-1, :]], axis=1)


def _shift_next(x):
    return jnp.concatenate([x[:, 1:, :], jnp.zeros_like(x[:, :1, :])], axis=1)


def _conv_bn_kernel(bold_ref, w1a_ref, w1b_ref, b1a_ref, b1b_ref,
                    g1a_ref, g1b_ref, h1a_ref, h1b_ref,
                    w2a_ref, w2b_ref, c2b_ref, g2_ref, h2_ref, out_ref):
    x = jnp.nan_to_num(bold_ref[...])
    xp, xn = _shift_prev(x), _shift_next(x)
    a = w1a_ref[0] * xp + w1a_ref[1] * x + w1a_ref[2] * xn + b1a_ref[...]
    b = w1b_ref[0] * xp + w1b_ref[1] * x + w1b_ref[2] * xn + b1b_ref[...]

    def bn(y, g, h):
        m = jnp.mean(y, axis=(0, 1), keepdims=True)
        v = jnp.mean((y - m) ** 2, axis=(0, 1), keepdims=True)
        return (y - m) * lax.rsqrt(v + EPS) * g + h

    a = jnp.maximum(bn(a, g1a_ref[...], h1a_ref[...]), 0.0)
    b = jnp.maximum(bn(b, g1b_ref[...], h1b_ref[...]), 0.0)
    ap, an = _shift_prev(a), _shift_next(a)
    bp, bnx = _shift_prev(b), _shift_next(b)
    y = (w2a_ref[0] * ap + w2a_ref[1] * a + w2a_ref[2] * an
         + w2b_ref[0] * bp + w2b_ref[1] * b + w2b_ref[2] * bnx + c2b_ref[...])
    out_ref[...] = jnp.maximum(bn(y, g2_ref[...], h2_ref[...]), 0.0)


def _ln(x, g, h):
    m = jnp.mean(x, axis=-1, keepdims=True)
    v = jnp.mean((x - m) ** 2, axis=-1, keepdims=True)
    return (x - m) * lax.rsqrt(v + EPS) * g + h


def _softmax(x):
    m = jnp.max(x, axis=-1, keepdims=True)
    e = jnp.exp(x - m)
    return e / jnp.sum(e, axis=-1, keepdims=True)


def _dot_t(x, w):
    return lax.dot_general(x.astype(jnp.bfloat16), w.astype(jnp.bfloat16),
                           (((1,), (1,)), ((), ())),
                           preferred_element_type=jnp.float32)


def _dot(x, w):
    return lax.dot_general(x.astype(jnp.bfloat16), w.astype(jnp.bfloat16),
                           (((1,), (0,)), ((), ())),
                           preferred_element_type=jnp.float32)


_EG = 8
DHP = 128
BF = jnp.bfloat16


def _mm_t(a, b):
    return lax.dot_general(a, b, (((1,), (1,)), ((), ())),
                           preferred_element_type=jnp.float32)


def _mm(a, b):
    return lax.dot_general(a, b, (((1,), (0,)), ((), ())),
                           preferred_element_type=jnp.float32)


def _encoder_kernel(x2_ref, tpw_ref, tpb_ref, wq_ref, wqb_ref, wk_ref, wkb_ref,
                    ln1g_ref, ln1b_ref, qw_ref, qb_ref, kw_ref, kb_ref,
                    vw_ref, vb_ref, outw_ref, outb_ref,
                    ln2g_ref, ln2b_ref, ff1w_ref, ff1b_ref, ff2w_ref, ff2b_ref,
                    pw_ref, pb_ref, lg_ref, lb_ref, qt_ref, o_ref):
    G = _EG
    scale_a = DH ** -0.5
    tpw = tpw_ref[...].astype(BF)
    hs = [lax.dot_general(x2_ref[g].astype(BF), tpw,
                          (((0,), (1,)), ((), ())),
                          preferred_element_type=jnp.float32)
          for g in range(G)]
    h = jnp.concatenate(hs, axis=0) + tpb_ref[...]
    hb = h.astype(BF)
    q = (_mm_t(hb, wq_ref[...].astype(BF)) + wqb_ref[...]).astype(BF)
    k = (_mm_t(hb, wk_ref[...].astype(BF)) + wkb_ref[...]).astype(BF)
    zs = []
    for g in range(G):
        sg = slice(g * R, (g + 1) * R)
        adj = _softmax(_mm_t(q[sg], k[sg]) * (D ** -0.5)).astype(BF)
        zs.append(_mm(adj, hb[sg]))
    z = jnp.concatenate(zs, axis=0)
    for l in range(NL):
        yb = _ln(z, ln1g_ref[l], ln1b_ref[l]).astype(BF)
        qp = (_mm_t(yb, qw_ref[l].astype(BF)) + qb_ref[l]).astype(BF)
        kp = (_mm_t(yb, kw_ref[l].astype(BF)) + kb_ref[l]).astype(BF)
        vp = (_mm_t(yb, vw_ref[l].astype(BF)) + vb_ref[l]).astype(BF)
        os_ = []
        for g in range(G):
            sg = slice(g * R, (g + 1) * R)
            for hh in range(H):
                hsl = slice(hh * DHP, (hh + 1) * DHP)
                s = _mm_t(qp[sg, hsl], kp[sg, hsl]) * scale_a
                p = _softmax(s).astype(BF)
                os_.append(_mm(p, vp[sg, hsl]).astype(BF))
        o = jnp.concatenate(
            [jnp.concatenate(os_[g * H:(g + 1) * H], axis=1)
             for g in range(G)], axis=0)
        z = z + _mm_t(o, outw_ref[l].astype(BF)) + outb_ref[l]
        y2b = _ln(z, ln2g_ref[l], ln2b_ref[l]).astype(BF)
        f = jnp.maximum(_mm_t(y2b, ff1w_ref[l].astype(BF)) + ff1b_ref[l], 0.0)
        z = z + _mm_t(f.astype(BF), ff2w_ref[l].astype(BF)) + ff2b_ref[l]
    qtb = qt_ref[0].astype(BF)
    zb = z.astype(BF)
    ctxs = []
    for g in range(G):
        sg = slice(g * R, (g + 1) * R)
        att = _softmax(_mm_t(qtb, zb[sg]) * (D ** -0.5)).astype(BF)
        ctxs.append(_mm(att, zb[sg]))
    ctx = jnp.concatenate(ctxs, axis=0).astype(BF)
    br = _ln(_mm_t(ctx, pw_ref[...].astype(BF)) + pb_ref[...],
             lg_ref[...], lb_ref[...])
    for g in range(G):
        o_ref[g] = br[g * NQ:(g + 1) * NQ]


_GB = 2


def _gather_kernel(ids_ref, brain_ref, table_ref, out_ref, sem_r):
    b0 = pl.program_id(0) * _GB
    for g in range(_GB):
        for s in range(S):
            tok = ids_ref[b0 + g, s]
            pltpu.make_async_copy(
                table_ref.at[tok], out_ref.at[g, NQ + s], sem_r).start()
    for g in range(_GB):
        out_ref[g, :NQ, :] = brain_ref[g]
    wait_cp = pltpu.make_async_copy(
        table_ref.at[0], out_ref.at[0, NQ], sem_r)
    for s in range(_GB * S):
        wait_cp.wait()


def kernel(bold, input_ids, attention_mask, labels,
           conv1_w, conv1_b, bn1_g, bn1_b, conv2_w, conv2_b, bn2_g, bn2_b,
           tproj_w, tproj_b, wq_w, wq_b, wk_w, wk_b,
           enc_ln1_g, enc_ln1_b, enc_qkv_w, enc_qkv_b, enc_out_w, enc_out_b,
           enc_ln2_g, enc_ln2_b, enc_ff1_w, enc_ff1_b, enc_ff2_w, enc_ff2_b,
           proj_w, proj_b, lnf_g, lnf_b, query_tokens, embed_table):
    f32 = jnp.float32
    w1 = conv1_w.reshape(R, 2, 3)
    w1a = w1[:, 0, :].transpose(1, 0).reshape(3, 1, R)
    w1b = w1[:, 1, :].transpose(1, 0).reshape(3, 1, R)
    b1 = conv1_b.reshape(R, 2)
    b1a, b1b = b1[:, 0].reshape(1, 1, R), b1[:, 1].reshape(1, 1, R)
    g1 = bn1_g.reshape(R, 2)
    g1a, g1b = g1[:, 0].reshape(1, 1, R), g1[:, 1].reshape(1, 1, R)
    h1 = bn1_b.reshape(R, 2)
    h1a, h1b = h1[:, 0].reshape(1, 1, R), h1[:, 1].reshape(1, 1, R)
    w2a = conv2_w[:, 0, :].transpose(1, 0).reshape(3, 1, R)
    w2b = conv2_w[:, 1, :].transpose(1, 0).reshape(3, 1, R)
    c2b = conv2_b.reshape(1, 1, R)
    g2, h2 = bn2_g.reshape(1, 1, R), bn2_b.reshape(1, 1, R)

    x2 = pl.pallas_call(
        _conv_bn_kernel,
        out_shape=jax.ShapeDtypeStruct((B, T, R), f32),
        name="conv_bn",
    )(bold, w1a, w1b, b1a, b1b, g1a, g1b, h1a, h1b, w2a, w2b, c2b, g2, h2)

    full = lambda shape: pl.BlockSpec(shape, lambda b: (0,) * len(shape))
    enc_in_specs = [
        pl.BlockSpec((_EG, T, R), lambda b: (b, 0, 0)),
        full((D, T)), full((1, D)),
        full((D, D)), full((1, D)), full((D, D)), full((1, D)),
        full((NL, D)), full((NL, D)),
        full((NL, H * DHP, D)), full((NL, H * DHP)),
        full((NL, H * DHP, D)), full((NL, H * DHP)),
        full((NL, H * DHP, D)), full((NL, H * DHP)),
        full((NL, D, H * DHP)), full((NL, D)),
        full((NL, D)), full((NL, D)),
        full((NL, FF, D)), full((NL, FF)),
        full((NL, D, FF)), full((NL, D)),
        full((HID, D)), full((1, HID)), full((1, HID)), full((1, HID)),
        full((1, NQ, D)),
    ]
    def _pad_heads_rows(w):
        return jnp.pad(w.reshape(NL, H, DH, D),
                       ((0, 0), (0, 0), (0, DHP - DH), (0, 0))
                       ).reshape(NL, H * DHP, D)

    def _pad_heads_bias(bv):
        return jnp.pad(bv.reshape(NL, H, DH),
                       ((0, 0), (0, 0), (0, DHP - DH))).reshape(NL, H * DHP)

    qw_p = _pad_heads_rows(enc_qkv_w[:, :D, :])
    kw_p = _pad_heads_rows(enc_qkv_w[:, D:2 * D, :])
    vw_p = _pad_heads_rows(enc_qkv_w[:, 2 * D:, :])
    qb_p = _pad_heads_bias(enc_qkv_b[:, :D])
    kb_p = _pad_heads_bias(enc_qkv_b[:, D:2 * D])
    vb_p = _pad_heads_bias(enc_qkv_b[:, 2 * D:])
    outw_p = jnp.pad(enc_out_w.reshape(NL, D, H, DH),
                     ((0, 0), (0, 0), (0, 0), (0, DHP - DH))
                     ).reshape(NL, D, H * DHP)

    brain = pl.pallas_call(
        _encoder_kernel,
        grid=(B // _EG,),
        in_specs=enc_in_specs,
        out_specs=pl.BlockSpec((_EG, NQ, HID), lambda b: (b, 0, 0)),
        out_shape=jax.ShapeDtypeStruct((B, NQ, HID), f32),
        compiler_params=pltpu.CompilerParams(
            dimension_semantics=("parallel",),
        ),
        name="encoder",
    )(x2, tproj_w, tproj_b.reshape(1, D),
      wq_w, wq_b.reshape(1, D), wk_w, wk_b.reshape(1, D),
      enc_ln1_g, enc_ln1_b, qw_p, qb_p, kw_p, kb_p, vw_p, vb_p,
      outw_p, enc_out_b,
      enc_ln2_g, enc_ln2_b, enc_ff1_w, enc_ff1_b, enc_ff2_w, enc_ff2_b,
      proj_w, proj_b.reshape(1, HID), lnf_g.reshape(1, HID),
      lnf_b.reshape(1, HID), query_tokens)

    inputs_embeds = pl.pallas_call(
        _gather_kernel,
        grid_spec=pltpu.PrefetchScalarGridSpec(
            num_scalar_prefetch=1,
            grid=(B // _GB,),
            in_specs=[pl.BlockSpec((_GB, NQ, HID), lambda b, ids: (b, 0, 0)),
                      pl.BlockSpec(memory_space=pl.ANY)],
            out_specs=pl.BlockSpec((_GB, NQ + S, HID),
                                   lambda b, ids: (b, 0, 0)),
            scratch_shapes=[pltpu.SemaphoreType.DMA],
        ),
        out_shape=jax.ShapeDtypeStruct((B, NQ + S, HID), f32),
        compiler_params=pltpu.CompilerParams(
            dimension_semantics=("arbitrary",),
            vmem_limit_bytes=50 * 1024 * 1024,
        ),
        name="gather_concat",
    )(input_ids.astype(jnp.int32), brain, embed_table)

    full_mask = jnp.concatenate(
        [jnp.ones((B, NQ), attention_mask.dtype), attention_mask], axis=1)
    full_labels = jnp.concatenate(
        [jnp.full((B, NQ), -100, labels.dtype), labels], axis=1)
    return inputs_embeds, full_mask, full_labels
